# baseline (speedup 1.0000x reference)
import dataclasses
import functools

import numpy as np
import jax
import jax.numpy as jnp
from jax import lax
from jax.experimental import pallas as pl
from jax.experimental.pallas import tpu as pltpu
from jax.experimental.pallas import tpu_sc as plsc

D = 128
HW = D // 2

SC_CORES = 2
SC_TILES = 16
BATCH = 40

_PREC = lax.Precision.HIGHEST


def _build_dft() -> np.ndarray:
    j = np.arange(D)[:, None].astype(np.float64)
    k = np.arange(HW + 1)[None, :].astype(np.float64)
    ang = 2.0 * np.pi * j * k / D
    cos, msin = np.cos(ang), -np.sin(ang)
    m = np.zeros((D, D), np.float64)
    m[:, 0:HW] = cos[:, 0:HW]
    m[:, HW] = cos[:, HW]
    m[:, HW + 1:] = msin[:, 1:HW]
    return m.astype(np.float32)


def _build_irfft() -> np.ndarray:
    n = np.arange(D)[None, :].astype(np.float64)
    k = np.arange(HW + 1)[:, None].astype(np.float64)
    w = np.where((k == 0) | (k == HW), 1.0, 2.0)
    ang = 2.0 * np.pi * n * k / D
    tc, ts = w * np.cos(ang) / D, -w * np.sin(ang) / D
    m = np.zeros((D, D), np.float64)
    m[0:HW] = tc[0:HW]
    m[HW] = tc[HW]
    m[HW + 1:] = ts[1:HW]
    return m.astype(np.float32)


_DFT = _build_dft()
_IRFFT = _build_irfft()


def _pre_body(x_ref, rel_ref, dft_ref, w_rel_ref, xf_ref, rf_ref, rel_out_ref):
    dft = dft_ref[...]
    xf_ref[...] = lax.dot(x_ref[...], dft, precision=_PREC)
    rf_ref[...] = lax.dot(rel_ref[...], dft, precision=_PREC)
    rel_out_ref[...] = lax.dot(rel_ref[...], w_rel_ref[...], precision=_PREC)


def _pre_call(x, rel, w_rel):
    n, r = x.shape[0], rel.shape[0]
    return pl.pallas_call(
        _pre_body,
        out_shape=[
            jax.ShapeDtypeStruct((n, D), jnp.float32),
            jax.ShapeDtypeStruct((r, D), jnp.float32),
            jax.ShapeDtypeStruct((r, D), jnp.float32),
        ],
    )(x, rel, jnp.asarray(_DFT), w_rel)


def _sc_body(n_pad, n_batches, xf_hbm, rf_hbm, src_hbm, dst_hbm, et_hbm,
             nrm_hbm, zeros_hbm, out_hbm, acc, src_v0, src_v1, et_v0, et_v1,
             dst_v0, dst_v1, dst_v2, dst_v3, nrm_v0, nrm_v1, a_v0, a_v1,
             rf_tile, z_v0, z_v1, sem_i0, sem_i1, sem_g0, sem_g1,
             sem_s0, sem_s1):
    cid = lax.axis_index("c")
    sid = lax.axis_index("s")
    rows = n_pad // SC_TILES
    src_v = (src_v0, src_v1)
    et_v = (et_v0, et_v1)
    dst_v = (dst_v0, dst_v1, dst_v2, dst_v3)
    nrm_v = (nrm_v0, nrm_v1)
    a_v = (a_v0, a_v1)
    z_v = (z_v0, z_v1)
    sem_i = (sem_i0, sem_i1)
    sem_g = (sem_g0, sem_g1)
    sem_s = (sem_s0, sem_s1)

    def _idx_copies(bi, p, d):
        return (
            pltpu.make_async_copy(src_hbm.at[cid, sid, bi], src_v[p], sem_i[p]),
            pltpu.make_async_copy(et_hbm.at[cid, sid, bi], et_v[p], sem_i[p]),
            pltpu.make_async_copy(dst_hbm.at[cid, sid, bi], dst_v[d], sem_i[p]),
            pltpu.make_async_copy(nrm_hbm.at[cid, sid, bi], nrm_v[p], sem_i[p]),
        )

    def _gather_copies(p):
        return (
            pltpu.make_async_copy(xf_hbm.at[src_v[p]], a_v[p], sem_g[p]),
        )

    def _scatter_start(p, d):
        pltpu.async_copy(z_v[p], acc.at[dst_v[d]], sem_s[p], add=True)

    def _scatter_wait(p, d):
        pltpu.make_async_copy(z_v[p], acc.at[dst_v[d]], sem_s[p]).wait()

    def _start(copies):
        for c in copies:
            c.start()

    def _wait(copies):
        for c in copies:
            c.wait()

    _start(_idx_copies(0, 0, 0))
    _start(_idx_copies(1, 1, 1))
    pltpu.sync_copy(zeros_hbm, acc.at[pl.ds(sid * rows, rows)])
    pltpu.sync_copy(rf_hbm, rf_tile)
    _wait(_idx_copies(0, 0, 0))
    _start(_gather_copies(0))
    plsc.subcore_barrier()

    def _process(bi, p, d):
        q = 1 - p
        _wait(_gather_copies(p))

        @pl.when(bi + 1 < n_batches)
        def _launch_next_gather():
            _wait(_idx_copies(bi + 1, q, (d + 1) % 4))
            _start(_gather_copies(q))

        @pl.when(bi >= 2)
        def _drain_scatter():
            _scatter_wait(p, (d + 2) % 4)

        av, zv = a_v[p], z_v[p]
        iota16 = lax.iota(jnp.int32, 16)
        lane0 = iota16 == 0
        cols = [iota16 + k * 16 for k in range(D // 16)]

        @pl.loop(0, BATCH)
        def _edge(i):
            splat_i = jnp.full((16,), 0, jnp.int32) + i
            vn = plsc.load_gather(nrm_v[p], [splat_i])
            etb = plsc.load_gather(et_v[p], [splat_i])
            for c in range(HW // 16):
                lo = pl.ds(c * 16, 16)
                hi = pl.ds(HW + c * 16, 16)
                ar = av[i, lo]
                ai = av[i, hi]
                br = plsc.load_gather(rf_tile, [etb, cols[c]])
                bi_ = plsc.load_gather(rf_tile, [etb, cols[HW // 16 + c]])
                rr = ar * br
                ii = ai * bi_
                zre = rr + ii
                zim = ar * bi_ - ai * br
                if c == 0:
                    zre = jnp.where(lane0, rr, zre)
                    zim = jnp.where(lane0, ii, zim)
                zv[i, lo] = zre * vn
                zv[i, hi] = zim * vn

        _scatter_start(p, d)

        @pl.when(bi + 2 < n_batches)
        def _prefetch_idx():
            _start(_idx_copies(bi + 2, p, (d + 2) % 4))

    @pl.loop(0, n_batches, step=4)
    def _quad(b0):
        _process(b0, 0, 0)
        for k in range(1, 4):
            @pl.when(b0 + k < n_batches)
            def _go(k=k):
                _process(b0 + k, k % 2, k)

    _scatter_wait(0, (n_batches - 2) % 4)
    _scatter_wait(1, (n_batches - 1) % 4)
    plsc.subcore_barrier()
    pltpu.sync_copy(acc.at[pl.ds(sid * rows, rows)],
                    out_hbm.at[cid, pl.ds(sid * rows, rows)])


def _sc_call(xf, rf, src, dst, et, nrm):
    n = xf.shape[0]
    e = src.shape[0]
    half = e // 2
    per_tile = half // SC_TILES
    n_batches = per_tile // BATCH
    assert half * 2 == e and per_tile * SC_TILES == half
    assert n_batches * BATCH == per_tile
    n_pad = -(-n // (SC_TILES * 8)) * (SC_TILES * 8)

    cp = pltpu.CompilerParams()
    if "needs_layout_passes" in pltpu.CompilerParams.__dataclass_fields__:
        cp = dataclasses.replace(cp, needs_layout_passes=False)
    shape4 = (SC_CORES, SC_TILES, n_batches, BATCH)
    kern = pl.kernel(
        functools.partial(_sc_body, n_pad, n_batches),
        out_type=jax.ShapeDtypeStruct((SC_CORES, n_pad, D), jnp.float32),
        mesh=plsc.VectorSubcoreMesh(core_axis_name="c", subcore_axis_name="s"),
        compiler_params=cp,
        scratch_types=[
            pltpu.VMEM_SHARED((n_pad, D), jnp.float32),
            pltpu.VMEM((BATCH,), jnp.int32),
            pltpu.VMEM((BATCH,), jnp.int32),
            pltpu.VMEM((BATCH,), jnp.int32),
            pltpu.VMEM((BATCH,), jnp.int32),
            pltpu.VMEM((BATCH,), jnp.int32),
            pltpu.VMEM((BATCH,), jnp.int32),
            pltpu.VMEM((BATCH,), jnp.int32),
            pltpu.VMEM((BATCH,), jnp.int32),
            pltpu.VMEM((BATCH,), jnp.float32),
            pltpu.VMEM((BATCH,), jnp.float32),
            pltpu.VMEM((BATCH, D), jnp.float32),
            pltpu.VMEM((BATCH, D), jnp.float32),
            pltpu.VMEM((rf.shape[0], D), jnp.float32),
            pltpu.VMEM((BATCH, D), jnp.float32),
            pltpu.VMEM((BATCH, D), jnp.float32),
            pltpu.SemaphoreType.DMA,
            pltpu.SemaphoreType.DMA,
            pltpu.SemaphoreType.DMA,
            pltpu.SemaphoreType.DMA,
            pltpu.SemaphoreType.DMA,
            pltpu.SemaphoreType.DMA,
        ],
    )
    zeros = jnp.zeros((n_pad // SC_TILES, D), jnp.float32)
    z = kern(xf, rf, src.reshape(shape4), dst.reshape(shape4),
             et.reshape(shape4), nrm.reshape(shape4), zeros)
    return z[:, :n, :]


def _post_body(nblk, z0_ref, z1_ref, xf_ref, irfft_ref, dft_ref, in_w_ref,
               out_w_ref, loop_w_ref, loop_rel_ref, bias_ref, gamma_ref,
               beta_ref, out_ref, pre_ref, stats_ref):
    p = pl.program_id(0)
    j = pl.program_id(1)
    blk = out_ref.shape[0]

    @pl.when(p == 0)
    def _compute():
        t = irfft_ref[...]
        g_in = lax.dot(t, in_w_ref[...], precision=_PREC)
        g_out = lax.dot(t, out_w_ref[...], precision=_PREC)
        g_loop = lax.dot(t, loop_w_ref[...], precision=_PREC)
        lf = lax.dot(loop_rel_ref[...], dft_ref[...], precision=_PREC)
        lre = lf[:, :HW]
        lim = lf[:, HW:]
        xf = xf_ref[...]
        are = xf[:, :HW]
        aim = xf[:, HW:]
        zre = are * lre + aim * lim
        zim = are * lim - aim * lre
        col0 = lax.broadcasted_iota(jnp.int32, (1, HW), 1) == 0
        zre = jnp.where(col0, are * lre, zre)
        zim = jnp.where(col0, aim * lim, zim)
        zl = jnp.concatenate([zre, zim], axis=1)
        pre = (lax.dot(z0_ref[...], g_in, precision=_PREC)
               + lax.dot(z1_ref[...], g_out, precision=_PREC)
               + lax.dot(zl, g_loop, precision=_PREC)) / 3.0 + bias_ref[...]
        pre_ref[pl.ds(j * blk, blk), :] = pre

        @pl.when(j == 0)
        def _init():
            stats_ref[...] = jnp.zeros_like(stats_ref)

        stats = jnp.concatenate(
            [jnp.sum(pre, axis=0, keepdims=True),
             jnp.sum(pre * pre, axis=0, keepdims=True),
             jnp.zeros((6, D), jnp.float32)], axis=0)
        stats_ref[...] += stats

    @pl.when(p == 1)
    def _normalize():
        n_total = jnp.float32(pre_ref.shape[0])
        mean = stats_ref[0, :] / n_total
        var = stats_ref[1, :] / n_total - mean * mean
        scale = lax.rsqrt(var + 1e-5) * gamma_ref[0, :]
        shift = beta_ref[0, :] - mean * scale
        out_ref[...] = pre_ref[pl.ds(j * blk, blk), :] * scale + shift


def _post_call(z0, z1, xf, in_w, out_w, loop_w, loop_rel, bias, gamma, beta):
    n = xf.shape[0]
    nblk = 10
    blk = n // nblk
    assert blk * nblk == n

    row_spec = pl.BlockSpec((blk, D), lambda p, j: (j, 0))
    full = lambda shape: pl.BlockSpec(shape, lambda p, j: (0, 0))
    return pl.pallas_call(
        functools.partial(_post_body, nblk),
        grid=(2, nblk),
        in_specs=[
            row_spec, row_spec, row_spec,
            full((D, D)), full((D, D)), full((D, D)), full((D, D)),
            full((D, D)), full((1, D)), full((1, D)), full((1, D)),
            full((1, D)),
        ],
        out_specs=pl.BlockSpec((blk, D), lambda p, j: (j, 0)),
        out_shape=jax.ShapeDtypeStruct((n, D), jnp.float32),
        scratch_shapes=[
            pltpu.VMEM((n, D), jnp.float32),
            pltpu.VMEM((8, D), jnp.float32),
        ],
    )(z0, z1, xf, jnp.asarray(_IRFFT), jnp.asarray(_DFT), in_w, out_w,
      loop_w, loop_rel, bias.reshape(1, D), gamma.reshape(1, D),
      beta.reshape(1, D))


def kernel(x, edge_index, rel_repr, edge_type, edge_norm, in_w, out_w,
           loop_w, w_rel, loop_rel, bias, bn_gamma, bn_beta):
    src = edge_index[0]
    dst = edge_index[1]
    xf, rf, rel_out = _pre_call(x, rel_repr, w_rel)
    z = _sc_call(xf, rf, src, dst, edge_type, edge_norm)
    out = _post_call(z[0], z[1], xf, in_w, out_w, loop_w, loop_rel, bias,
                     bn_gamma, bn_beta)
    return out, rel_out


# speedup vs baseline: 1.0189x; 1.0189x over previous
import dataclasses
import functools

import numpy as np
import jax
import jax.numpy as jnp
from jax import lax
from jax.experimental import pallas as pl
from jax.experimental.pallas import tpu as pltpu
from jax.experimental.pallas import tpu_sc as plsc

D = 128
HW = D // 2

SC_CORES = 2
SC_TILES = 16
BATCH = 40

_PREC = lax.Precision.HIGHEST


def _build_dft() -> np.ndarray:
    j = np.arange(D)[:, None].astype(np.float64)
    k = np.arange(HW + 1)[None, :].astype(np.float64)
    ang = 2.0 * np.pi * j * k / D
    cos, msin = np.cos(ang), -np.sin(ang)
    m = np.zeros((D, D), np.float64)
    m[:, 0:HW] = cos[:, 0:HW]
    m[:, HW] = cos[:, HW]
    m[:, HW + 1:] = msin[:, 1:HW]
    return m.astype(np.float32)


def _build_irfft() -> np.ndarray:
    n = np.arange(D)[None, :].astype(np.float64)
    k = np.arange(HW + 1)[:, None].astype(np.float64)
    w = np.where((k == 0) | (k == HW), 1.0, 2.0)
    ang = 2.0 * np.pi * n * k / D
    tc, ts = w * np.cos(ang) / D, -w * np.sin(ang) / D
    m = np.zeros((D, D), np.float64)
    m[0:HW] = tc[0:HW]
    m[HW] = tc[HW]
    m[HW + 1:] = ts[1:HW]
    return m.astype(np.float32)


_DFT = _build_dft()
_IRFFT = _build_irfft()


def _pre_body(x_ref, rel_ref, dft_ref, w_rel_ref, xf_ref, rf_ref, rel_out_ref):
    dft = dft_ref[...]
    xf_ref[...] = lax.dot(x_ref[...], dft, precision=_PREC)
    rf_ref[...] = lax.dot(rel_ref[...], dft, precision=_PREC)
    rel_out_ref[...] = lax.dot(rel_ref[...], w_rel_ref[...], precision=_PREC)


def _pre_call(x, rel, w_rel):
    n, r = x.shape[0], rel.shape[0]
    return pl.pallas_call(
        _pre_body,
        out_shape=[
            jax.ShapeDtypeStruct((n, D), jnp.float32),
            jax.ShapeDtypeStruct((r, D), jnp.float32),
            jax.ShapeDtypeStruct((r, D), jnp.float32),
        ],
    )(x, rel, jnp.asarray(_DFT), w_rel)


def _sc_body(n_pad, n_batches, xf_hbm, rf_hbm, idx_hbm, zeros_hbm, out_hbm,
             acc, idx_v0, idx_v1, idx_v2, idx_v3, a_v0, a_v1,
             b_v0, b_v1, z_v0, z_v1, sem_i0, sem_i1, sem_g0, sem_g1,
             sem_s0, sem_s1):
    cid = lax.axis_index("c")
    sid = lax.axis_index("s")
    rows = n_pad // SC_TILES
    idx_v = (idx_v0, idx_v1, idx_v2, idx_v3)
    a_v = (a_v0, a_v1)
    b_v = (b_v0, b_v1)
    z_v = (z_v0, z_v1)
    sem_i = (sem_i0, sem_i1)
    sem_g = (sem_g0, sem_g1)
    sem_s = (sem_s0, sem_s1)

    def _idx_copies(bi, p, d):
        return (
            pltpu.make_async_copy(idx_hbm.at[cid, sid, bi], idx_v[d], sem_i[p]),
        )

    def _gather_copies(p, d):
        return (
            pltpu.make_async_copy(xf_hbm.at[idx_v[d].at[0]], a_v[p], sem_g[p]),
            pltpu.make_async_copy(rf_hbm.at[idx_v[d].at[1]], b_v[p], sem_g[p]),
        )

    def _scatter_start(p, d):
        pltpu.async_copy(z_v[p], acc.at[idx_v[d].at[2]], sem_s[p], add=True)

    def _scatter_wait(p, d):
        pltpu.make_async_copy(z_v[p], acc.at[idx_v[d].at[2]], sem_s[p]).wait()

    def _start(copies):
        for c in copies:
            c.start()

    def _wait(copies):
        for c in copies:
            c.wait()

    _start(_idx_copies(0, 0, 0))
    _start(_idx_copies(1, 1, 1))
    pltpu.sync_copy(zeros_hbm, acc.at[pl.ds(sid * rows, rows)])
    _wait(_idx_copies(0, 0, 0))
    _start(_gather_copies(0, 0))
    plsc.subcore_barrier()

    def _process(bi, p, d):
        q = 1 - p
        _wait(_gather_copies(p, d))

        @pl.when(bi + 1 < n_batches)
        def _launch_next_gather():
            _wait(_idx_copies(bi + 1, q, (d + 1) % 4))
            _start(_gather_copies(q, (d + 1) % 4))

        @pl.when(bi >= 2)
        def _drain_scatter():
            _scatter_wait(p, (d + 2) % 4)

        av, bv, zv = a_v[p], b_v[p], z_v[p]
        idxd = idx_v[d]

        @pl.loop(0, BATCH)
        def _edge(i):
            splat_i = jnp.full((16,), 0, jnp.int32) + i
            vn = plsc.bitcast(
                plsc.load_gather(idxd, [jnp.full((16,), 3, jnp.int32),
                                        splat_i]), jnp.float32)
            lane0 = lax.iota(jnp.int32, 16) == 0
            for c in range(HW // 16):
                lo = pl.ds(c * 16, 16)
                hi = pl.ds(HW + c * 16, 16)
                ar = av[i, lo]
                ai = av[i, hi]
                br = bv[i, lo]
                bi_ = bv[i, hi]
                rr = ar * br
                ii = ai * bi_
                zre = rr + ii
                zim = ar * bi_ - ai * br
                if c == 0:
                    zre = jnp.where(lane0, rr, zre)
                    zim = jnp.where(lane0, ii, zim)
                zv[i, lo] = zre * vn
                zv[i, hi] = zim * vn

        _scatter_start(p, d)

        @pl.when(bi + 2 < n_batches)
        def _prefetch_idx():
            _start(_idx_copies(bi + 2, p, (d + 2) % 4))

    @pl.loop(0, n_batches, step=4)
    def _quad(b0):
        _process(b0, 0, 0)
        for k in range(1, 4):
            @pl.when(b0 + k < n_batches)
            def _go(k=k):
                _process(b0 + k, k % 2, k)

    _scatter_wait(0, (n_batches - 2) % 4)
    _scatter_wait(1, (n_batches - 1) % 4)
    plsc.subcore_barrier()
    pltpu.sync_copy(acc.at[pl.ds(sid * rows, rows)],
                    out_hbm.at[cid, pl.ds(sid * rows, rows)])


def _sc_call(xf, rf, src, dst, et, nrm):
    n = xf.shape[0]
    e = src.shape[0]
    half = e // 2
    per_tile = half // SC_TILES
    n_batches = per_tile // BATCH
    assert half * 2 == e and per_tile * SC_TILES == half
    assert n_batches * BATCH == per_tile
    n_pad = -(-n // (SC_TILES * 8)) * (SC_TILES * 8)

    cp = pltpu.CompilerParams()
    if "needs_layout_passes" in pltpu.CompilerParams.__dataclass_fields__:
        cp = dataclasses.replace(cp, needs_layout_passes=False)
    shape4 = (SC_CORES, SC_TILES, n_batches, BATCH)
    kern = pl.kernel(
        functools.partial(_sc_body, n_pad, n_batches),
        out_type=jax.ShapeDtypeStruct((SC_CORES, n_pad, D), jnp.float32),
        mesh=plsc.VectorSubcoreMesh(core_axis_name="c", subcore_axis_name="s"),
        compiler_params=cp,
        scratch_types=[
            pltpu.VMEM_SHARED((n_pad, D), jnp.float32),
            pltpu.VMEM((4, BATCH), jnp.int32),
            pltpu.VMEM((4, BATCH), jnp.int32),
            pltpu.VMEM((4, BATCH), jnp.int32),
            pltpu.VMEM((4, BATCH), jnp.int32),
            pltpu.VMEM((BATCH, D), jnp.float32),
            pltpu.VMEM((BATCH, D), jnp.float32),
            pltpu.VMEM((BATCH, D), jnp.float32),
            pltpu.VMEM((BATCH, D), jnp.float32),
            pltpu.VMEM((BATCH, D), jnp.float32),
            pltpu.VMEM((BATCH, D), jnp.float32),
            pltpu.SemaphoreType.DMA,
            pltpu.SemaphoreType.DMA,
            pltpu.SemaphoreType.DMA,
            pltpu.SemaphoreType.DMA,
            pltpu.SemaphoreType.DMA,
            pltpu.SemaphoreType.DMA,
        ],
    )
    zeros = jnp.zeros((n_pad // SC_TILES, D), jnp.float32)
    nrm_i = lax.bitcast_convert_type(nrm, jnp.int32)
    idx_pack = jnp.stack(
        [src.reshape(shape4), et.reshape(shape4), dst.reshape(shape4),
         nrm_i.reshape(shape4)], axis=3)
    z = kern(xf, rf, idx_pack, zeros)
    return z[:, :n, :]


def _post_body(nblk, z0_ref, z1_ref, xf_ref, irfft_ref, dft_ref, in_w_ref,
               out_w_ref, loop_w_ref, loop_rel_ref, bias_ref, gamma_ref,
               beta_ref, out_ref, pre_ref, stats_ref):
    p = pl.program_id(0)
    j = pl.program_id(1)
    blk = out_ref.shape[0]

    @pl.when(p == 0)
    def _compute():
        t = irfft_ref[...]
        g_in = lax.dot(t, in_w_ref[...], precision=_PREC)
        g_out = lax.dot(t, out_w_ref[...], precision=_PREC)
        g_loop = lax.dot(t, loop_w_ref[...], precision=_PREC)
        lf = lax.dot(loop_rel_ref[...], dft_ref[...], precision=_PREC)
        lre = lf[:, :HW]
        lim = lf[:, HW:]
        xf = xf_ref[...]
        are = xf[:, :HW]
        aim = xf[:, HW:]
        zre = are * lre + aim * lim
        zim = are * lim - aim * lre
        col0 = lax.broadcasted_iota(jnp.int32, (1, HW), 1) == 0
        zre = jnp.where(col0, are * lre, zre)
        zim = jnp.where(col0, aim * lim, zim)
        zl = jnp.concatenate([zre, zim], axis=1)
        pre = (lax.dot(z0_ref[...], g_in, precision=_PREC)
               + lax.dot(z1_ref[...], g_out, precision=_PREC)
               + lax.dot(zl, g_loop, precision=_PREC)) / 3.0 + bias_ref[...]
        pre_ref[pl.ds(j * blk, blk), :] = pre

        @pl.when(j == 0)
        def _init():
            stats_ref[...] = jnp.zeros_like(stats_ref)

        stats = jnp.concatenate(
            [jnp.sum(pre, axis=0, keepdims=True),
             jnp.sum(pre * pre, axis=0, keepdims=True),
             jnp.zeros((6, D), jnp.float32)], axis=0)
        stats_ref[...] += stats

    @pl.when(p == 1)
    def _normalize():
        n_total = jnp.float32(pre_ref.shape[0])
        mean = stats_ref[0, :] / n_total
        var = stats_ref[1, :] / n_total - mean * mean
        scale = lax.rsqrt(var + 1e-5) * gamma_ref[0, :]
        shift = beta_ref[0, :] - mean * scale
        out_ref[...] = pre_ref[pl.ds(j * blk, blk), :] * scale + shift


def _post_call(z0, z1, xf, in_w, out_w, loop_w, loop_rel, bias, gamma, beta):
    n = xf.shape[0]
    nblk = 10
    blk = n // nblk
    assert blk * nblk == n

    row_spec = pl.BlockSpec((blk, D), lambda p, j: (j, 0))
    full = lambda shape: pl.BlockSpec(shape, lambda p, j: (0, 0))
    return pl.pallas_call(
        functools.partial(_post_body, nblk),
        grid=(2, nblk),
        in_specs=[
            row_spec, row_spec, row_spec,
            full((D, D)), full((D, D)), full((D, D)), full((D, D)),
            full((D, D)), full((1, D)), full((1, D)), full((1, D)),
            full((1, D)),
        ],
        out_specs=pl.BlockSpec((blk, D), lambda p, j: (j, 0)),
        out_shape=jax.ShapeDtypeStruct((n, D), jnp.float32),
        scratch_shapes=[
            pltpu.VMEM((n, D), jnp.float32),
            pltpu.VMEM((8, D), jnp.float32),
        ],
    )(z0, z1, xf, jnp.asarray(_IRFFT), jnp.asarray(_DFT), in_w, out_w,
      loop_w, loop_rel, bias.reshape(1, D), gamma.reshape(1, D),
      beta.reshape(1, D))


def kernel(x, edge_index, rel_repr, edge_type, edge_norm, in_w, out_w,
           loop_w, w_rel, loop_rel, bias, bn_gamma, bn_beta):
    src = edge_index[0]
    dst = edge_index[1]
    xf, rf, rel_out = _pre_call(x, rel_repr, w_rel)
    z = _sc_call(xf, rf, src, dst, edge_type, edge_norm)
    out = _post_call(z[0], z[1], xf, in_w, out_w, loop_w, loop_rel, bias,
                     bn_gamma, bn_beta)
    return out, rel_out


# speedup vs baseline: 1.2260x; 1.2033x over previous
import dataclasses
import functools

import numpy as np
import jax
import jax.numpy as jnp
from jax import lax
from jax.experimental import pallas as pl
from jax.experimental.pallas import tpu as pltpu
from jax.experimental.pallas import tpu_sc as plsc

D = 128
HW = D // 2

SC_CORES = 2
SC_TILES = 16
BATCH = 80

_PREC = lax.Precision.HIGHEST


def _build_dft() -> np.ndarray:
    j = np.arange(D)[:, None].astype(np.float64)
    k = np.arange(HW + 1)[None, :].astype(np.float64)
    ang = 2.0 * np.pi * j * k / D
    cos, msin = np.cos(ang), -np.sin(ang)
    m = np.zeros((D, D), np.float64)
    m[:, 0:HW] = cos[:, 0:HW]
    m[:, HW] = cos[:, HW]
    m[:, HW + 1:] = msin[:, 1:HW]
    return m.astype(np.float32)


def _build_irfft() -> np.ndarray:
    n = np.arange(D)[None, :].astype(np.float64)
    k = np.arange(HW + 1)[:, None].astype(np.float64)
    w = np.where((k == 0) | (k == HW), 1.0, 2.0)
    ang = 2.0 * np.pi * n * k / D
    tc, ts = w * np.cos(ang) / D, -w * np.sin(ang) / D
    m = np.zeros((D, D), np.float64)
    m[0:HW] = tc[0:HW]
    m[HW] = tc[HW]
    m[HW + 1:] = ts[1:HW]
    return m.astype(np.float32)


_DFT = _build_dft()
_IRFFT = _build_irfft()


def _pre_body(x_ref, rel_ref, dft_ref, w_rel_ref, xf_ref, rf_ref, rel_out_ref):
    dft = dft_ref[...]
    xf_ref[...] = lax.dot(x_ref[...], dft, precision=_PREC)
    rf_ref[...] = lax.dot(rel_ref[...], dft, precision=_PREC)
    rel_out_ref[...] = lax.dot(rel_ref[...], w_rel_ref[...], precision=_PREC)


def _pre_call(x, rel, w_rel):
    n, r = x.shape[0], rel.shape[0]
    return pl.pallas_call(
        _pre_body,
        out_shape=[
            jax.ShapeDtypeStruct((n, D), jnp.float32),
            jax.ShapeDtypeStruct((r, D), jnp.float32),
            jax.ShapeDtypeStruct((r, D), jnp.float32),
        ],
    )(x, rel, jnp.asarray(_DFT), w_rel)


def _sc_body(n_pad, n_batches, xf_hbm, rf_hbm, src_hbm, dst_hbm, et_hbm,
             nrm_hbm, zeros_hbm, out_hbm, acc, src_v0, src_v1, et_v0, et_v1,
             dst_v0, dst_v1, dst_v2, dst_v3, nrm_v0, nrm_v1, a_v0, a_v1,
             b_v0, b_v1, sem_i0, sem_i1, sem_g0, sem_g1):
    cid = lax.axis_index("c")
    sid = lax.axis_index("s")
    rows = n_pad // SC_TILES
    src_v = (src_v0, src_v1)
    et_v = (et_v0, et_v1)
    dst_v = (dst_v0, dst_v1, dst_v2, dst_v3)
    nrm_v = (nrm_v0, nrm_v1)
    a_v = (a_v0, a_v1)
    b_v = (b_v0, b_v1)
    sem_i = (sem_i0, sem_i1)
    sem_g = (sem_g0, sem_g1)

    def _idx_copies(bi, p, d):
        return (
            pltpu.make_async_copy(src_hbm.at[cid, sid, bi], src_v[p], sem_i[p]),
            pltpu.make_async_copy(et_hbm.at[cid, sid, bi], et_v[p], sem_i[p]),
            pltpu.make_async_copy(dst_hbm.at[cid, sid, bi], dst_v[d], sem_i[p]),
            pltpu.make_async_copy(nrm_hbm.at[cid, sid, bi], nrm_v[p], sem_i[p]),
        )

    def _gather_copies(p):
        return (
            pltpu.make_async_copy(xf_hbm.at[src_v[p]], a_v[p], sem_g[p]),
            pltpu.make_async_copy(rf_hbm.at[et_v[p]], b_v[p], sem_g[p]),
        )

    def _start(copies):
        for c in copies:
            c.start()

    def _wait(copies):
        for c in copies:
            c.wait()

    _start(_idx_copies(0, 0, 0))
    _start(_idx_copies(1, 1, 1))
    pltpu.sync_copy(zeros_hbm, acc.at[pl.ds(sid * rows, rows)])
    _wait(_idx_copies(0, 0, 0))
    _start(_gather_copies(0))
    plsc.subcore_barrier()

    def _process(bi, p, d):
        q = 1 - p
        _wait(_gather_copies(p))

        @pl.when(bi + 1 < n_batches)
        def _launch_next_gather():
            _wait(_idx_copies(bi + 1, q, (d + 1) % 4))
            _start(_gather_copies(q))

        av, bv = a_v[p], b_v[p]

        @pl.loop(0, BATCH)
        def _edge(i):
            vn = plsc.load_gather(nrm_v[p], [jnp.full((16,), 0, jnp.int32) + i])
            lane0 = lax.iota(jnp.int32, 16) == 0
            for c in range(HW // 16):
                lo = pl.ds(c * 16, 16)
                hi = pl.ds(HW + c * 16, 16)
                ar = av[i, lo]
                ai = av[i, hi]
                br = bv[i, lo]
                bi_ = bv[i, hi]
                rr = ar * br
                ii = ai * bi_
                zre = rr + ii
                zim = ar * bi_ - ai * br
                if c == 0:
                    zre = jnp.where(lane0, rr, zre)
                    zim = jnp.where(lane0, ii, zim)
                av[i, lo] = zre * vn
                av[i, hi] = zim * vn

        pltpu.sync_copy(av, acc.at[dst_v[d]], add=True)

        @pl.when(bi + 2 < n_batches)
        def _prefetch_idx():
            _start(_idx_copies(bi + 2, p, (d + 2) % 4))

    @pl.loop(0, n_batches, step=4)
    def _quad(b0):
        _process(b0, 0, 0)
        for k in range(1, 4):
            @pl.when(b0 + k < n_batches)
            def _go(k=k):
                _process(b0 + k, k % 2, k)

    plsc.subcore_barrier()
    pltpu.sync_copy(acc.at[pl.ds(sid * rows, rows)],
                    out_hbm.at[cid, pl.ds(sid * rows, rows)])


def _sc_call(xf, rf, src, dst, et, nrm):
    n = xf.shape[0]
    e = src.shape[0]
    half = e // 2
    per_tile = half // SC_TILES
    n_batches = per_tile // BATCH
    assert half * 2 == e and per_tile * SC_TILES == half
    assert n_batches * BATCH == per_tile
    n_pad = -(-n // (SC_TILES * 8)) * (SC_TILES * 8)

    cp = pltpu.CompilerParams()
    if "needs_layout_passes" in pltpu.CompilerParams.__dataclass_fields__:
        cp = dataclasses.replace(cp, needs_layout_passes=False)
    shape4 = (SC_CORES, SC_TILES, n_batches, BATCH)
    kern = pl.kernel(
        functools.partial(_sc_body, n_pad, n_batches),
        out_type=jax.ShapeDtypeStruct((SC_CORES, n_pad, D), jnp.float32),
        mesh=plsc.VectorSubcoreMesh(core_axis_name="c", subcore_axis_name="s"),
        compiler_params=cp,
        scratch_types=[
            pltpu.VMEM_SHARED((n_pad, D), jnp.float32),
            pltpu.VMEM((BATCH,), jnp.int32),
            pltpu.VMEM((BATCH,), jnp.int32),
            pltpu.VMEM((BATCH,), jnp.int32),
            pltpu.VMEM((BATCH,), jnp.int32),
            pltpu.VMEM((BATCH,), jnp.int32),
            pltpu.VMEM((BATCH,), jnp.int32),
            pltpu.VMEM((BATCH,), jnp.int32),
            pltpu.VMEM((BATCH,), jnp.int32),
            pltpu.VMEM((BATCH,), jnp.float32),
            pltpu.VMEM((BATCH,), jnp.float32),
            pltpu.VMEM((BATCH, D), jnp.float32),
            pltpu.VMEM((BATCH, D), jnp.float32),
            pltpu.VMEM((BATCH, D), jnp.float32),
            pltpu.VMEM((BATCH, D), jnp.float32),
            pltpu.SemaphoreType.DMA,
            pltpu.SemaphoreType.DMA,
            pltpu.SemaphoreType.DMA,
            pltpu.SemaphoreType.DMA,
        ],
    )
    zeros = jnp.zeros((n_pad // SC_TILES, D), jnp.float32)
    z = kern(xf, rf, src.reshape(shape4), dst.reshape(shape4),
             et.reshape(shape4), nrm.reshape(shape4), zeros)
    return z[:, :n, :]


def _post_body(nblk, z0_ref, z1_ref, xf_ref, irfft_ref, dft_ref, in_w_ref,
               out_w_ref, loop_w_ref, loop_rel_ref, bias_ref, gamma_ref,
               beta_ref, out_ref, pre_ref, stats_ref):
    p = pl.program_id(0)
    j = pl.program_id(1)
    blk = out_ref.shape[0]

    @pl.when(p == 0)
    def _compute():
        t = irfft_ref[...]
        g_in = lax.dot(t, in_w_ref[...], precision=_PREC)
        g_out = lax.dot(t, out_w_ref[...], precision=_PREC)
        g_loop = lax.dot(t, loop_w_ref[...], precision=_PREC)
        lf = lax.dot(loop_rel_ref[...], dft_ref[...], precision=_PREC)
        lre = lf[:, :HW]
        lim = lf[:, HW:]
        xf = xf_ref[...]
        are = xf[:, :HW]
        aim = xf[:, HW:]
        zre = are * lre + aim * lim
        zim = are * lim - aim * lre
        col0 = lax.broadcasted_iota(jnp.int32, (1, HW), 1) == 0
        zre = jnp.where(col0, are * lre, zre)
        zim = jnp.where(col0, aim * lim, zim)
        zl = jnp.concatenate([zre, zim], axis=1)
        pre = (lax.dot(z0_ref[...], g_in, precision=_PREC)
               + lax.dot(z1_ref[...], g_out, precision=_PREC)
               + lax.dot(zl, g_loop, precision=_PREC)) / 3.0 + bias_ref[...]
        pre_ref[pl.ds(j * blk, blk), :] = pre

        @pl.when(j == 0)
        def _init():
            stats_ref[...] = jnp.zeros_like(stats_ref)

        stats = jnp.concatenate(
            [jnp.sum(pre, axis=0, keepdims=True),
             jnp.sum(pre * pre, axis=0, keepdims=True),
             jnp.zeros((6, D), jnp.float32)], axis=0)
        stats_ref[...] += stats

    @pl.when(p == 1)
    def _normalize():
        n_total = jnp.float32(pre_ref.shape[0])
        mean = stats_ref[0, :] / n_total
        var = stats_ref[1, :] / n_total - mean * mean
        scale = lax.rsqrt(var + 1e-5) * gamma_ref[0, :]
        shift = beta_ref[0, :] - mean * scale
        out_ref[...] = pre_ref[pl.ds(j * blk, blk), :] * scale + shift


def _post_call(z0, z1, xf, in_w, out_w, loop_w, loop_rel, bias, gamma, beta):
    n = xf.shape[0]
    nblk = 10
    blk = n // nblk
    assert blk * nblk == n

    row_spec = pl.BlockSpec((blk, D), lambda p, j: (j, 0))
    full = lambda shape: pl.BlockSpec(shape, lambda p, j: (0, 0))
    return pl.pallas_call(
        functools.partial(_post_body, nblk),
        grid=(2, nblk),
        in_specs=[
            row_spec, row_spec, row_spec,
            full((D, D)), full((D, D)), full((D, D)), full((D, D)),
            full((D, D)), full((1, D)), full((1, D)), full((1, D)),
            full((1, D)),
        ],
        out_specs=pl.BlockSpec((blk, D), lambda p, j: (j, 0)),
        out_shape=jax.ShapeDtypeStruct((n, D), jnp.float32),
        scratch_shapes=[
            pltpu.VMEM((n, D), jnp.float32),
            pltpu.VMEM((8, D), jnp.float32),
        ],
    )(z0, z1, xf, jnp.asarray(_IRFFT), jnp.asarray(_DFT), in_w, out_w,
      loop_w, loop_rel, bias.reshape(1, D), gamma.reshape(1, D),
      beta.reshape(1, D))


def kernel(x, edge_index, rel_repr, edge_type, edge_norm, in_w, out_w,
           loop_w, w_rel, loop_rel, bias, bn_gamma, bn_beta):
    src = edge_index[0]
    dst = edge_index[1]
    xf, rf, rel_out = _pre_call(x, rel_repr, w_rel)
    z = _sc_call(xf, rf, src, dst, edge_type, edge_norm)
    out = _post_call(z[0], z[1], xf, in_w, out_w, loop_w, loop_rel, bias,
                     bn_gamma, bn_beta)
    return out, rel_out


# speedup vs baseline: 1.9767x; 1.6123x over previous
import dataclasses
import functools

import numpy as np
import jax
import jax.numpy as jnp
from jax import lax
from jax.experimental import pallas as pl
from jax.experimental.pallas import tpu as pltpu
from jax.experimental.pallas import tpu_sc as plsc

D = 128
HW = D // 2

SC_CORES = 2
SC_TILES = 16
BATCH = 80

_PREC = lax.Precision.HIGHEST


def _build_dft() -> np.ndarray:
    j = np.arange(D)[:, None].astype(np.float64)
    k = np.arange(HW + 1)[None, :].astype(np.float64)
    ang = 2.0 * np.pi * j * k / D
    cos, msin = np.cos(ang), -np.sin(ang)
    m = np.zeros((D, D), np.float64)
    m[:, 0:HW] = cos[:, 0:HW]
    m[:, HW] = cos[:, HW]
    m[:, HW + 1:] = msin[:, 1:HW]
    return m.astype(np.float32)


def _build_irfft() -> np.ndarray:
    n = np.arange(D)[None, :].astype(np.float64)
    k = np.arange(HW + 1)[:, None].astype(np.float64)
    w = np.where((k == 0) | (k == HW), 1.0, 2.0)
    ang = 2.0 * np.pi * n * k / D
    tc, ts = w * np.cos(ang) / D, -w * np.sin(ang) / D
    m = np.zeros((D, D), np.float64)
    m[0:HW] = tc[0:HW]
    m[HW] = tc[HW]
    m[HW + 1:] = ts[1:HW]
    return m.astype(np.float32)


_DFT = _build_dft()
_IRFFT = _build_irfft()


def _pre_body(x_ref, rel_ref, dft_ref, w_rel_ref, xf_ref, rf_ref, rel_out_ref):
    dft = dft_ref[...]
    xf_ref[...] = lax.dot(x_ref[...], dft, precision=_PREC)
    rf_ref[...] = lax.dot(rel_ref[...], dft, precision=_PREC)
    rel_out_ref[...] = lax.dot(rel_ref[...], w_rel_ref[...], precision=_PREC)


def _pre_call(x, rel, w_rel):
    n, r = x.shape[0], rel.shape[0]
    return pl.pallas_call(
        _pre_body,
        out_shape=[
            jax.ShapeDtypeStruct((n, D), jnp.float32),
            jax.ShapeDtypeStruct((r, D), jnp.float32),
            jax.ShapeDtypeStruct((r, D), jnp.float32),
        ],
    )(x, rel, jnp.asarray(_DFT), w_rel)


def _sc_body(n_pad, n_batches, xf_hbm, rf_hbm, src_hbm, dst_hbm, et_hbm,
             nrm_hbm, zeros_hbm, out_hbm, acc, src_v0, src_v1, et_v0, et_v1,
             dst_v0, dst_v1, dst_v2, dst_v3, nrm_v0, nrm_v1, a_v0, a_v1,
             b_v0, b_v1, sem_i0, sem_i1, sem_g0, sem_g1):
    cid = lax.axis_index("c")
    sid = lax.axis_index("s")
    rows = n_pad // SC_TILES
    src_v = (src_v0, src_v1)
    et_v = (et_v0, et_v1)
    dst_v = (dst_v0, dst_v1, dst_v2, dst_v3)
    nrm_v = (nrm_v0, nrm_v1)
    a_v = (a_v0, a_v1)
    b_v = (b_v0, b_v1)
    sem_i = (sem_i0, sem_i1)
    sem_g = (sem_g0, sem_g1)

    def _idx_copies(bi, p, d):
        return (
            pltpu.make_async_copy(src_hbm.at[cid, sid, bi], src_v[p], sem_i[p]),
            pltpu.make_async_copy(et_hbm.at[cid, sid, bi], et_v[p], sem_i[p]),
            pltpu.make_async_copy(dst_hbm.at[cid, sid, bi], dst_v[d], sem_i[p]),
            pltpu.make_async_copy(nrm_hbm.at[cid, sid, bi], nrm_v[p], sem_i[p]),
        )

    def _gather_copies(p):
        return (
            pltpu.make_async_copy(xf_hbm.at[src_v[p]], a_v[p], sem_g[p]),
            pltpu.make_async_copy(rf_hbm.at[et_v[p]], b_v[p], sem_g[p]),
        )

    def _start(copies):
        for c in copies:
            c.start()

    def _wait(copies):
        for c in copies:
            c.wait()

    _start(_idx_copies(0, 0, 0))
    _start(_idx_copies(1, 1, 1))
    pltpu.sync_copy(zeros_hbm, acc.at[pl.ds(sid * rows, rows)])
    _wait(_idx_copies(0, 0, 0))
    _start(_gather_copies(0))
    plsc.subcore_barrier()

    def _process(bi, p, d):
        q = 1 - p
        _wait(_gather_copies(p))

        @pl.when(bi + 1 < n_batches)
        def _launch_next_gather():
            _wait(_idx_copies(bi + 1, q, (d + 1) % 4))
            _start(_gather_copies(q))

        av, bv = a_v[p], b_v[p]

        @plsc.parallel_loop(0, BATCH, unroll=2)
        def _edge(i):
            vn = plsc.load_gather(nrm_v[p], [jnp.full((16,), 0, jnp.int32) + i])
            lane0 = lax.iota(jnp.int32, 16) == 0
            for c in range(HW // 16):
                lo = pl.ds(c * 16, 16)
                hi = pl.ds(HW + c * 16, 16)
                ar = av[i, lo]
                ai = av[i, hi]
                br = bv[i, lo]
                bi_ = bv[i, hi]
                rr = ar * br
                ii = ai * bi_
                zre = rr + ii
                zim = ar * bi_ - ai * br
                if c == 0:
                    zre = jnp.where(lane0, rr, zre)
                    zim = jnp.where(lane0, ii, zim)
                av[i, lo] = zre * vn
                av[i, hi] = zim * vn

        pltpu.sync_copy(av, acc.at[dst_v[d]], add=True)

        @pl.when(bi + 2 < n_batches)
        def _prefetch_idx():
            _start(_idx_copies(bi + 2, p, (d + 2) % 4))

    @pl.loop(0, n_batches, step=4)
    def _quad(b0):
        _process(b0, 0, 0)
        for k in range(1, 4):
            @pl.when(b0 + k < n_batches)
            def _go(k=k):
                _process(b0 + k, k % 2, k)

    plsc.subcore_barrier()
    pltpu.sync_copy(acc.at[pl.ds(sid * rows, rows)],
                    out_hbm.at[cid, pl.ds(sid * rows, rows)])


def _sc_call(xf, rf, src, dst, et, nrm):
    n = xf.shape[0]
    e = src.shape[0]
    half = e // 2
    per_tile = half // SC_TILES
    n_batches = per_tile // BATCH
    assert half * 2 == e and per_tile * SC_TILES == half
    assert n_batches * BATCH == per_tile
    n_pad = -(-n // (SC_TILES * 8)) * (SC_TILES * 8)

    cp = pltpu.CompilerParams()
    if "needs_layout_passes" in pltpu.CompilerParams.__dataclass_fields__:
        cp = dataclasses.replace(cp, needs_layout_passes=False)
    shape4 = (SC_CORES, SC_TILES, n_batches, BATCH)
    kern = pl.kernel(
        functools.partial(_sc_body, n_pad, n_batches),
        out_type=jax.ShapeDtypeStruct((SC_CORES, n_pad, D), jnp.float32),
        mesh=plsc.VectorSubcoreMesh(core_axis_name="c", subcore_axis_name="s"),
        compiler_params=cp,
        scratch_types=[
            pltpu.VMEM_SHARED((n_pad, D), jnp.float32),
            pltpu.VMEM((BATCH,), jnp.int32),
            pltpu.VMEM((BATCH,), jnp.int32),
            pltpu.VMEM((BATCH,), jnp.int32),
            pltpu.VMEM((BATCH,), jnp.int32),
            pltpu.VMEM((BATCH,), jnp.int32),
            pltpu.VMEM((BATCH,), jnp.int32),
            pltpu.VMEM((BATCH,), jnp.int32),
            pltpu.VMEM((BATCH,), jnp.int32),
            pltpu.VMEM((BATCH,), jnp.float32),
            pltpu.VMEM((BATCH,), jnp.float32),
            pltpu.VMEM((BATCH, D), jnp.float32),
            pltpu.VMEM((BATCH, D), jnp.float32),
            pltpu.VMEM((BATCH, D), jnp.float32),
            pltpu.VMEM((BATCH, D), jnp.float32),
            pltpu.SemaphoreType.DMA,
            pltpu.SemaphoreType.DMA,
            pltpu.SemaphoreType.DMA,
            pltpu.SemaphoreType.DMA,
        ],
    )
    zeros = jnp.zeros((n_pad // SC_TILES, D), jnp.float32)
    z = kern(xf, rf, src.reshape(shape4), dst.reshape(shape4),
             et.reshape(shape4), nrm.reshape(shape4), zeros)
    return z[:, :n, :]


def _post_body(nblk, z0_ref, z1_ref, xf_ref, irfft_ref, dft_ref, in_w_ref,
               out_w_ref, loop_w_ref, loop_rel_ref, bias_ref, gamma_ref,
               beta_ref, out_ref, pre_ref, stats_ref):
    p = pl.program_id(0)
    j = pl.program_id(1)
    blk = out_ref.shape[0]

    @pl.when(p == 0)
    def _compute():
        t = irfft_ref[...]
        g_in = lax.dot(t, in_w_ref[...], precision=_PREC)
        g_out = lax.dot(t, out_w_ref[...], precision=_PREC)
        g_loop = lax.dot(t, loop_w_ref[...], precision=_PREC)
        lf = lax.dot(loop_rel_ref[...], dft_ref[...], precision=_PREC)
        lre = lf[:, :HW]
        lim = lf[:, HW:]
        xf = xf_ref[...]
        are = xf[:, :HW]
        aim = xf[:, HW:]
        zre = are * lre + aim * lim
        zim = are * lim - aim * lre
        col0 = lax.broadcasted_iota(jnp.int32, (1, HW), 1) == 0
        zre = jnp.where(col0, are * lre, zre)
        zim = jnp.where(col0, aim * lim, zim)
        zl = jnp.concatenate([zre, zim], axis=1)
        pre = (lax.dot(z0_ref[...], g_in, precision=_PREC)
               + lax.dot(z1_ref[...], g_out, precision=_PREC)
               + lax.dot(zl, g_loop, precision=_PREC)) / 3.0 + bias_ref[...]
        pre_ref[pl.ds(j * blk, blk), :] = pre

        @pl.when(j == 0)
        def _init():
            stats_ref[...] = jnp.zeros_like(stats_ref)

        stats = jnp.concatenate(
            [jnp.sum(pre, axis=0, keepdims=True),
             jnp.sum(pre * pre, axis=0, keepdims=True),
             jnp.zeros((6, D), jnp.float32)], axis=0)
        stats_ref[...] += stats

    @pl.when(p == 1)
    def _normalize():
        n_total = jnp.float32(pre_ref.shape[0])
        mean = stats_ref[0, :] / n_total
        var = stats_ref[1, :] / n_total - mean * mean
        scale = lax.rsqrt(var + 1e-5) * gamma_ref[0, :]
        shift = beta_ref[0, :] - mean * scale
        out_ref[...] = pre_ref[pl.ds(j * blk, blk), :] * scale + shift


def _post_call(z0, z1, xf, in_w, out_w, loop_w, loop_rel, bias, gamma, beta):
    n = xf.shape[0]
    nblk = 10
    blk = n // nblk
    assert blk * nblk == n

    row_spec = pl.BlockSpec((blk, D), lambda p, j: (j, 0))
    full = lambda shape: pl.BlockSpec(shape, lambda p, j: (0, 0))
    return pl.pallas_call(
        functools.partial(_post_body, nblk),
        grid=(2, nblk),
        in_specs=[
            row_spec, row_spec, row_spec,
            full((D, D)), full((D, D)), full((D, D)), full((D, D)),
            full((D, D)), full((1, D)), full((1, D)), full((1, D)),
            full((1, D)),
        ],
        out_specs=pl.BlockSpec((blk, D), lambda p, j: (j, 0)),
        out_shape=jax.ShapeDtypeStruct((n, D), jnp.float32),
        scratch_shapes=[
            pltpu.VMEM((n, D), jnp.float32),
            pltpu.VMEM((8, D), jnp.float32),
        ],
    )(z0, z1, xf, jnp.asarray(_IRFFT), jnp.asarray(_DFT), in_w, out_w,
      loop_w, loop_rel, bias.reshape(1, D), gamma.reshape(1, D),
      beta.reshape(1, D))


def kernel(x, edge_index, rel_repr, edge_type, edge_norm, in_w, out_w,
           loop_w, w_rel, loop_rel, bias, bn_gamma, bn_beta):
    src = edge_index[0]
    dst = edge_index[1]
    xf, rf, rel_out = _pre_call(x, rel_repr, w_rel)
    z = _sc_call(xf, rf, src, dst, edge_type, edge_norm)
    out = _post_call(z[0], z[1], xf, in_w, out_w, loop_w, loop_rel, bias,
                     bn_gamma, bn_beta)
    return out, rel_out


# speedup vs baseline: 1.9780x; 1.0007x over previous
import dataclasses
import functools

import numpy as np
import jax
import jax.numpy as jnp
from jax import lax
from jax.experimental import pallas as pl
from jax.experimental.pallas import tpu as pltpu
from jax.experimental.pallas import tpu_sc as plsc

D = 128
HW = D // 2

SC_CORES = 2
SC_TILES = 16
BATCH = 80

_PREC = lax.Precision.HIGHEST


def _build_dft() -> np.ndarray:
    j = np.arange(D)[:, None].astype(np.float64)
    k = np.arange(HW + 1)[None, :].astype(np.float64)
    ang = 2.0 * np.pi * j * k / D
    cos, msin = np.cos(ang), -np.sin(ang)
    m = np.zeros((D, D), np.float64)
    m[:, 0:HW] = cos[:, 0:HW]
    m[:, HW] = cos[:, HW]
    m[:, HW + 1:] = msin[:, 1:HW]
    return m.astype(np.float32)


def _build_irfft() -> np.ndarray:
    n = np.arange(D)[None, :].astype(np.float64)
    k = np.arange(HW + 1)[:, None].astype(np.float64)
    w = np.where((k == 0) | (k == HW), 1.0, 2.0)
    ang = 2.0 * np.pi * n * k / D
    tc, ts = w * np.cos(ang) / D, -w * np.sin(ang) / D
    m = np.zeros((D, D), np.float64)
    m[0:HW] = tc[0:HW]
    m[HW] = tc[HW]
    m[HW + 1:] = ts[1:HW]
    return m.astype(np.float32)


_DFT = _build_dft()
_IRFFT = _build_irfft()


def _pre_body(x_ref, rel_ref, dft_ref, w_rel_ref, xf_ref, rf_ref, rel_out_ref):
    dft = dft_ref[...]
    xf_ref[...] = lax.dot(x_ref[...], dft, precision=_PREC)
    rf_ref[...] = lax.dot(rel_ref[...], dft, precision=_PREC)
    rel_out_ref[...] = lax.dot(rel_ref[...], w_rel_ref[...], precision=_PREC)


def _pre_call(x, rel, w_rel):
    n, r = x.shape[0], rel.shape[0]
    return pl.pallas_call(
        _pre_body,
        out_shape=[
            jax.ShapeDtypeStruct((n, D), jnp.float32),
            jax.ShapeDtypeStruct((r, D), jnp.float32),
            jax.ShapeDtypeStruct((r, D), jnp.float32),
        ],
    )(x, rel, jnp.asarray(_DFT), w_rel)


def _sc_body(n_pad, n_batches, xf_hbm, rf_hbm, src_hbm, dst_hbm, et_hbm,
             nrm_hbm, zeros_hbm, out_hbm, acc, src_v0, src_v1, et_v0, et_v1,
             dst_v0, dst_v1, dst_v2, dst_v3, nrm_v0, nrm_v1, a_v0, a_v1,
             b_v0, b_v1, sem_i0, sem_i1, sem_g0, sem_g1):
    cid = lax.axis_index("c")
    sid = lax.axis_index("s")
    rows = n_pad // SC_TILES
    src_v = (src_v0, src_v1)
    et_v = (et_v0, et_v1)
    dst_v = (dst_v0, dst_v1, dst_v2, dst_v3)
    nrm_v = (nrm_v0, nrm_v1)
    a_v = (a_v0, a_v1)
    b_v = (b_v0, b_v1)
    sem_i = (sem_i0, sem_i1)
    sem_g = (sem_g0, sem_g1)

    def _idx_copies(bi, p, d):
        return (
            pltpu.make_async_copy(src_hbm.at[cid, sid, bi], src_v[p], sem_i[p]),
            pltpu.make_async_copy(et_hbm.at[cid, sid, bi], et_v[p], sem_i[p]),
            pltpu.make_async_copy(dst_hbm.at[cid, sid, bi], dst_v[d], sem_i[p]),
            pltpu.make_async_copy(nrm_hbm.at[cid, sid, bi], nrm_v[p], sem_i[p]),
        )

    def _gather_copies(p):
        return (
            pltpu.make_async_copy(xf_hbm.at[src_v[p]], a_v[p], sem_g[p]),
            pltpu.make_async_copy(rf_hbm.at[et_v[p]], b_v[p], sem_g[p]),
        )

    def _start(copies):
        for c in copies:
            c.start()

    def _wait(copies):
        for c in copies:
            c.wait()

    _start(_idx_copies(0, 0, 0))
    _start(_idx_copies(1, 1, 1))
    pltpu.sync_copy(zeros_hbm, acc.at[pl.ds(sid * rows, rows)])
    _wait(_idx_copies(0, 0, 0))
    _start(_gather_copies(0))
    plsc.subcore_barrier()

    def _process(bi, p, d):
        q = 1 - p
        _wait(_gather_copies(p))

        @pl.when(bi + 1 < n_batches)
        def _launch_next_gather():
            _wait(_idx_copies(bi + 1, q, (d + 1) % 4))
            _start(_gather_copies(q))

        av, bv = a_v[p], b_v[p]

        @plsc.parallel_loop(0, BATCH, unroll=4)
        def _edge(i):
            vn = plsc.load_gather(nrm_v[p], [jnp.full((16,), 0, jnp.int32) + i])
            lane0 = lax.iota(jnp.int32, 16) == 0
            for c in range(HW // 16):
                lo = pl.ds(c * 16, 16)
                hi = pl.ds(HW + c * 16, 16)
                ar = av[i, lo]
                ai = av[i, hi]
                br = bv[i, lo]
                bi_ = bv[i, hi]
                rr = ar * br
                ii = ai * bi_
                zre = rr + ii
                zim = ar * bi_ - ai * br
                if c == 0:
                    zre = jnp.where(lane0, rr, zre)
                    zim = jnp.where(lane0, ii, zim)
                av[i, lo] = zre * vn
                av[i, hi] = zim * vn

        pltpu.sync_copy(av, acc.at[dst_v[d]], add=True)

        @pl.when(bi + 2 < n_batches)
        def _prefetch_idx():
            _start(_idx_copies(bi + 2, p, (d + 2) % 4))

    @pl.loop(0, n_batches, step=4)
    def _quad(b0):
        _process(b0, 0, 0)
        for k in range(1, 4):
            @pl.when(b0 + k < n_batches)
            def _go(k=k):
                _process(b0 + k, k % 2, k)

    plsc.subcore_barrier()
    pltpu.sync_copy(acc.at[pl.ds(sid * rows, rows)],
                    out_hbm.at[cid, pl.ds(sid * rows, rows)])


def _sc_call(xf, rf, src, dst, et, nrm):
    n = xf.shape[0]
    e = src.shape[0]
    half = e // 2
    per_tile = half // SC_TILES
    n_batches = per_tile // BATCH
    assert half * 2 == e and per_tile * SC_TILES == half
    assert n_batches * BATCH == per_tile
    n_pad = -(-n // (SC_TILES * 8)) * (SC_TILES * 8)

    cp = pltpu.CompilerParams()
    if "needs_layout_passes" in pltpu.CompilerParams.__dataclass_fields__:
        cp = dataclasses.replace(cp, needs_layout_passes=False)
    shape4 = (SC_CORES, SC_TILES, n_batches, BATCH)
    kern = pl.kernel(
        functools.partial(_sc_body, n_pad, n_batches),
        out_type=jax.ShapeDtypeStruct((SC_CORES, n_pad, D), jnp.float32),
        mesh=plsc.VectorSubcoreMesh(core_axis_name="c", subcore_axis_name="s"),
        compiler_params=cp,
        scratch_types=[
            pltpu.VMEM_SHARED((n_pad, D), jnp.float32),
            pltpu.VMEM((BATCH,), jnp.int32),
            pltpu.VMEM((BATCH,), jnp.int32),
            pltpu.VMEM((BATCH,), jnp.int32),
            pltpu.VMEM((BATCH,), jnp.int32),
            pltpu.VMEM((BATCH,), jnp.int32),
            pltpu.VMEM((BATCH,), jnp.int32),
            pltpu.VMEM((BATCH,), jnp.int32),
            pltpu.VMEM((BATCH,), jnp.int32),
            pltpu.VMEM((BATCH,), jnp.float32),
            pltpu.VMEM((BATCH,), jnp.float32),
            pltpu.VMEM((BATCH, D), jnp.float32),
            pltpu.VMEM((BATCH, D), jnp.float32),
            pltpu.VMEM((BATCH, D), jnp.float32),
            pltpu.VMEM((BATCH, D), jnp.float32),
            pltpu.SemaphoreType.DMA,
            pltpu.SemaphoreType.DMA,
            pltpu.SemaphoreType.DMA,
            pltpu.SemaphoreType.DMA,
        ],
    )
    zeros = jnp.zeros((n_pad // SC_TILES, D), jnp.float32)
    z = kern(xf, rf, src.reshape(shape4), dst.reshape(shape4),
             et.reshape(shape4), nrm.reshape(shape4), zeros)
    return z[:, :n, :]


def _post_body(nblk, z0_ref, z1_ref, xf_ref, irfft_ref, dft_ref, in_w_ref,
               out_w_ref, loop_w_ref, loop_rel_ref, bias_ref, gamma_ref,
               beta_ref, out_ref, pre_ref, stats_ref):
    p = pl.program_id(0)
    j = pl.program_id(1)
    blk = out_ref.shape[0]

    @pl.when(p == 0)
    def _compute():
        t = irfft_ref[...]
        g_in = lax.dot(t, in_w_ref[...], precision=_PREC)
        g_out = lax.dot(t, out_w_ref[...], precision=_PREC)
        g_loop = lax.dot(t, loop_w_ref[...], precision=_PREC)
        lf = lax.dot(loop_rel_ref[...], dft_ref[...], precision=_PREC)
        lre = lf[:, :HW]
        lim = lf[:, HW:]
        xf = xf_ref[...]
        are = xf[:, :HW]
        aim = xf[:, HW:]
        zre = are * lre + aim * lim
        zim = are * lim - aim * lre
        col0 = lax.broadcasted_iota(jnp.int32, (1, HW), 1) == 0
        zre = jnp.where(col0, are * lre, zre)
        zim = jnp.where(col0, aim * lim, zim)
        zl = jnp.concatenate([zre, zim], axis=1)
        pre = (lax.dot(z0_ref[...], g_in, precision=_PREC)
               + lax.dot(z1_ref[...], g_out, precision=_PREC)
               + lax.dot(zl, g_loop, precision=_PREC)) / 3.0 + bias_ref[...]
        pre_ref[pl.ds(j * blk, blk), :] = pre

        @pl.when(j == 0)
        def _init():
            stats_ref[...] = jnp.zeros_like(stats_ref)

        stats = jnp.concatenate(
            [jnp.sum(pre, axis=0, keepdims=True),
             jnp.sum(pre * pre, axis=0, keepdims=True),
             jnp.zeros((6, D), jnp.float32)], axis=0)
        stats_ref[...] += stats

    @pl.when(p == 1)
    def _normalize():
        n_total = jnp.float32(pre_ref.shape[0])
        mean = stats_ref[0, :] / n_total
        var = stats_ref[1, :] / n_total - mean * mean
        scale = lax.rsqrt(var + 1e-5) * gamma_ref[0, :]
        shift = beta_ref[0, :] - mean * scale
        out_ref[...] = pre_ref[pl.ds(j * blk, blk), :] * scale + shift


def _post_call(z0, z1, xf, in_w, out_w, loop_w, loop_rel, bias, gamma, beta):
    n = xf.shape[0]
    nblk = 10
    blk = n // nblk
    assert blk * nblk == n

    row_spec = pl.BlockSpec((blk, D), lambda p, j: (j, 0))
    full = lambda shape: pl.BlockSpec(shape, lambda p, j: (0, 0))
    return pl.pallas_call(
        functools.partial(_post_body, nblk),
        grid=(2, nblk),
        in_specs=[
            row_spec, row_spec, row_spec,
            full((D, D)), full((D, D)), full((D, D)), full((D, D)),
            full((D, D)), full((1, D)), full((1, D)), full((1, D)),
            full((1, D)),
        ],
        out_specs=pl.BlockSpec((blk, D), lambda p, j: (j, 0)),
        out_shape=jax.ShapeDtypeStruct((n, D), jnp.float32),
        scratch_shapes=[
            pltpu.VMEM((n, D), jnp.float32),
            pltpu.VMEM((8, D), jnp.float32),
        ],
    )(z0, z1, xf, jnp.asarray(_IRFFT), jnp.asarray(_DFT), in_w, out_w,
      loop_w, loop_rel, bias.reshape(1, D), gamma.reshape(1, D),
      beta.reshape(1, D))


def kernel(x, edge_index, rel_repr, edge_type, edge_norm, in_w, out_w,
           loop_w, w_rel, loop_rel, bias, bn_gamma, bn_beta):
    src = edge_index[0]
    dst = edge_index[1]
    xf, rf, rel_out = _pre_call(x, rel_repr, w_rel)
    z = _sc_call(xf, rf, src, dst, edge_type, edge_norm)
    out = _post_call(z[0], z[1], xf, in_w, out_w, loop_w, loop_rel, bias,
                     bn_gamma, bn_beta)
    return out, rel_out
